# Initial kernel scaffold; baseline (speedup 1.0000x reference)
import functools

import jax
import jax.numpy as jnp
from jax import lax
from jax.experimental import pallas as pl
from jax.experimental.pallas import tpu as pltpu
from jax.experimental.pallas import tpu_sc as plsc

NUM_EMB = 1_000_000
DIM = 64
ROWS = 16384
COLS = 50
B = ROWS * COLS

CHUNK = 128
MCHUNK = 4
SUPER = CHUNK * MCHUNK

_info = plsc.get_sparse_core_info()
NC, NS = _info.num_cores, _info.num_subcores
NW = NC * NS
PER_W = B // NW
N_CHUNKS = PER_W // CHUNK
N_SUPER = PER_W // SUPER

_mesh = plsc.VectorSubcoreMesh(core_axis_name="c", subcore_axis_name="s")


@functools.partial(
    pl.kernel,
    mesh=_mesh,
    out_type=jax.ShapeDtypeStruct((B, DIM), jnp.float32),
    scratch_types=[
        pltpu.VMEM((N_CHUNKS, CHUNK), jnp.int32),
        pltpu.VMEM((SUPER, DIM), jnp.float32),
        pltpu.SemaphoreType.DMA,
    ],
)
def _emb_lookup(idx_hbm, table_hbm, out_hbm, idx_v, rows_v, sem):
    wid = lax.axis_index("s") * NC + lax.axis_index("c")
    base = wid * PER_W
    pltpu.sync_copy(idx_hbm.at[wid], idx_v)

    def body(s, carry):
        copies = []
        for m in range(MCHUNK):
            c = s * MCHUNK + m
            copies.append(pltpu.async_copy(
                table_hbm.at[idx_v.at[c]],
                rows_v.at[pl.ds(m * CHUNK, CHUNK)],
                sem,
            ))
        for cp in copies:
            cp.wait()
        pltpu.sync_copy(rows_v, out_hbm.at[pl.ds(base + s * SUPER, SUPER)])
        return carry

    lax.fori_loop(0, N_SUPER, body, 0)


def kernel(token_ids, weight):
    idx = token_ids.reshape(NW, N_CHUNKS, CHUNK).astype(jnp.int32)
    out = _emb_lookup(idx, weight)
    return out.reshape(ROWS, COLS, DIM)


# speedup vs baseline: 1.8339x; 1.8339x over previous
import functools

import jax
import jax.numpy as jnp
from jax import lax
from jax.experimental import pallas as pl
from jax.experimental.pallas import tpu as pltpu
from jax.experimental.pallas import tpu_sc as plsc

NUM_EMB = 1_000_000
DIM = 64
ROWS = 16384
COLS = 50
B = ROWS * COLS

CHUNK = 128
MCHUNK = 4
SUPER = CHUNK * MCHUNK

_info = plsc.get_sparse_core_info()
NC, NS = _info.num_cores, _info.num_subcores
NW = NC * NS
PER_W = B // NW
N_CHUNKS = PER_W // CHUNK
N_SUPER = PER_W // SUPER

_mesh = plsc.VectorSubcoreMesh(core_axis_name="c", subcore_axis_name="s")


@functools.partial(
    pl.kernel,
    mesh=_mesh,
    out_type=jax.ShapeDtypeStruct((B, DIM), jnp.float32),
    scratch_types=[
        pltpu.VMEM((N_CHUNKS, CHUNK), jnp.int32),
        pltpu.VMEM((SUPER, DIM), jnp.float32),
        pltpu.SemaphoreType.DMA,
    ],
    compiler_params=pltpu.CompilerParams(use_tc_tiling_on_sc=False),
)
def _emb_lookup(idx_hbm, table_hbm, out_hbm, idx_v, rows_v, sem):
    wid = lax.axis_index("s") * NC + lax.axis_index("c")
    base = wid * PER_W
    pltpu.sync_copy(idx_hbm.at[wid], idx_v)

    def body(s, carry):
        copies = []
        for m in range(MCHUNK):
            c = s * MCHUNK + m
            copies.append(pltpu.async_copy(
                table_hbm.at[idx_v.at[c]],
                rows_v.at[pl.ds(m * CHUNK, CHUNK)],
                sem,
            ))
        for cp in copies:
            cp.wait()
        pltpu.sync_copy(rows_v, out_hbm.at[pl.ds(base + s * SUPER, SUPER)])
        return carry

    lax.fori_loop(0, N_SUPER, body, 0)


def kernel(token_ids, weight):
    idx = token_ids.reshape(NW, N_CHUNKS, CHUNK).astype(jnp.int32)
    out = _emb_lookup(idx, weight)
    return out.reshape(ROWS, COLS, DIM)


# speedup vs baseline: 1.8735x; 1.0216x over previous
import functools

import jax
import jax.numpy as jnp
from jax import lax
from jax.experimental import pallas as pl
from jax.experimental.pallas import tpu as pltpu
from jax.experimental.pallas import tpu_sc as plsc

NUM_EMB = 1_000_000
DIM = 64
ROWS = 16384
COLS = 50
B = ROWS * COLS

CHUNK = 128
MCHUNK = 4
SUPER = CHUNK * MCHUNK

_info = plsc.get_sparse_core_info()
NC, NS = _info.num_cores, _info.num_subcores
NW = NC * NS
PER_W = B // NW
N_CHUNKS = PER_W // CHUNK
N_SUPER = PER_W // SUPER

_mesh = plsc.VectorSubcoreMesh(core_axis_name="c", subcore_axis_name="s")


@functools.partial(
    pl.kernel,
    mesh=_mesh,
    out_type=jax.ShapeDtypeStruct((B, DIM), jnp.float32),
    scratch_types=[
        pltpu.VMEM((N_CHUNKS, CHUNK), jnp.int32),
        pltpu.VMEM((SUPER, DIM), jnp.float32),
        pltpu.VMEM((SUPER, DIM), jnp.float32),
        pltpu.SemaphoreType.DMA,
        pltpu.SemaphoreType.DMA,
        pltpu.SemaphoreType.DMA,
        pltpu.SemaphoreType.DMA,
    ],
    compiler_params=pltpu.CompilerParams(use_tc_tiling_on_sc=False),
)
def _emb_lookup(idx_hbm, table_hbm, out_hbm, idx_v, rows0, rows1,
                sem_g0, sem_g1, sem_w0, sem_w1):
    wid = lax.axis_index("s") * NC + lax.axis_index("c")
    base = wid * PER_W
    rows = (rows0, rows1)
    sem_g = (sem_g0, sem_g1)
    sem_w = (sem_w0, sem_w1)

    pltpu.sync_copy(idx_hbm.at[wid], idx_v)

    def fire(s, b):
        for m in range(MCHUNK):
            pltpu.async_copy(
                table_hbm.at[idx_v.at[s * MCHUNK + m]],
                rows[b].at[pl.ds(m * CHUNK, CHUNK)],
                sem_g[b],
            )

    def drain_gathers(b):
        pltpu.make_async_copy(out_hbm.at[pl.ds(0, SUPER)], rows[b],
                              sem_g[b]).wait()

    def wb_start(s, b):
        pltpu.async_copy(rows[b], out_hbm.at[pl.ds(base + s * SUPER, SUPER)],
                         sem_w[b])

    def wb_wait(b):
        pltpu.make_async_copy(rows[b], out_hbm.at[pl.ds(0, SUPER)],
                              sem_w[b]).wait()

    fire(0, 0)
    fire(1, 1)
    drain_gathers(0)
    wb_start(0, 0)

    def body(g, carry):
        for k in range(2):
            s = 1 + 2 * g + k
            b = 1 - k
            wb_wait(1 - b)
            fire(s + 1, 1 - b)
            drain_gathers(b)
            wb_start(s, b)
        return carry

    lax.fori_loop(0, (N_SUPER - 2) // 2, body, 0)

    wb_wait(0)
    drain_gathers(1)
    wb_start(N_SUPER - 1, 1)
    wb_wait(1)


def kernel(token_ids, weight):
    idx = token_ids.reshape(NW, N_CHUNKS, CHUNK).astype(jnp.int32)
    out = _emb_lookup(idx, weight)
    return out.reshape(ROWS, COLS, DIM)
